# baseline (speedup 1.0000x reference)
import functools

import jax
import jax.numpy as jnp
from jax import lax
from jax.experimental import pallas as pl
from jax.experimental.pallas import tpu as pltpu
from jax.experimental.pallas import tpu_sc as plsc

VOCAB = 1000000
EMB = 64
BATCH = 16384
NEGS = 15

NC = 2
NS = 16
L = 16
NW = NC * NS
BPW = BATCH // NW
G = BPW // L
NEG_G = NEGS * L


def _sc_dots_kernel(center_hbm, context_hbm, noise_hbm, mask_hbm,
                    wc_hbm, wx_hbm, out_hbm,
                    cidx_v, xidx_v, nidx_v, cen_v, ctx_v, msk_v, neg_v,
                    dots_v, sem):
    wid = lax.axis_index("s") * NC + lax.axis_index("c")
    base = pl.multiple_of(wid * BPW, BPW)

    pltpu.sync_copy(center_hbm.at[pl.ds(base, BPW)], cidx_v)
    pltpu.sync_copy(context_hbm.at[pl.ds(base, BPW)], xidx_v)
    pltpu.sync_copy(noise_hbm.at[wid], nidx_v)

    iota = lax.iota(jnp.int32, L)
    neg_rows = [iota * NEGS + j for j in range(NEGS)]

    def group(g, carry):
        row0 = pl.multiple_of(g * L, L)
        cidx = cidx_v[pl.ds(row0, L)]
        xidx = xidx_v[pl.ds(row0, L)]
        pltpu.async_copy(wc_hbm.at[cidx], cen_v, sem).wait()
        pltpu.async_copy(wx_hbm.at[xidx], ctx_v, sem).wait()
        pltpu.async_copy(wx_hbm.at[nidx_v.at[g]], neg_v, sem).wait()
        pltpu.sync_copy(mask_hbm.at[pl.ds(base + row0, L)], msk_v)

        def dbody(d, accs):
            dv = jnp.full((L,), d, dtype=jnp.int32)
            cen_d = plsc.load_gather(cen_v, [iota, dv])
            msk_d = plsc.load_gather(msk_v, [iota, dv])
            ctx_d = plsc.load_gather(ctx_v, [iota, dv])
            cm = cen_d * msk_d
            new = [accs[0] + cm * ctx_d]
            for j in range(NEGS):
                neg_d = plsc.load_gather(neg_v, [neg_rows[j], dv])
                new.append(accs[j + 1] + cm * neg_d)
            return tuple(new)

        zero = jnp.zeros((L,), jnp.float32)
        accs = lax.fori_loop(0, EMB, dbody, (zero,) * (1 + NEGS))

        out_rows = row0 + iota
        for k in range(1 + NEGS):
            plsc.store_scatter(dots_v, [out_rows, jnp.full((L,), k, jnp.int32)],
                               accs[k])
        return carry

    lax.fori_loop(0, G, group, 0)
    pltpu.sync_copy(dots_v, out_hbm.at[pl.ds(base, BPW)])


@functools.partial(
    pl.kernel,
    out_type=jax.ShapeDtypeStruct((BATCH, 1 + NEGS), jnp.float32),
    mesh=plsc.VectorSubcoreMesh(core_axis_name="c", subcore_axis_name="s"),
    compiler_params=pltpu.CompilerParams(needs_layout_passes=False,
                                         use_tc_tiling_on_sc=False),
    scratch_types=[
        pltpu.VMEM((BPW,), jnp.int32),
        pltpu.VMEM((BPW,), jnp.int32),
        pltpu.VMEM((G, NEG_G), jnp.int32),
        pltpu.VMEM((L, EMB), jnp.float32),
        pltpu.VMEM((L, EMB), jnp.float32),
        pltpu.VMEM((L, EMB), jnp.float32),
        pltpu.VMEM((NEG_G, EMB), jnp.float32),
        pltpu.VMEM((BPW, 1 + NEGS), jnp.float32),
        pltpu.SemaphoreType.DMA,
    ],
)
def _sc_dots(center, context, noise, mask, wc, wx, out,
             cidx_v, xidx_v, nidx_v, cen_v, ctx_v, msk_v, neg_v, dots_v, sem):
    _sc_dots_kernel(center, context, noise, mask, wc, wx, out,
                    cidx_v, xidx_v, nidx_v, cen_v, ctx_v, msk_v, neg_v,
                    dots_v, sem)


def _tc_loss_kernel(dots_ref, o_ref):
    x = dots_ref[...]
    col = lax.broadcasted_iota(jnp.int32, x.shape, 1)
    z = jnp.where(col == 0, x, -x)
    ls = jnp.minimum(z, 0.0) - jnp.log1p(jnp.exp(-jnp.abs(z)))
    o_ref[0, 0] = -jnp.sum(ls) / jnp.float32(BATCH)


def kernel(center, context, W_center, W_context):
    dk = jax.random.key(123)
    keep = jax.random.bernoulli(jax.random.fold_in(dk, 0), 0.9, (BATCH, EMB))
    mask_scale = jnp.where(keep, jnp.float32(1.0 / 0.9), jnp.float32(0.0))
    noise_idx = jax.random.randint(jax.random.fold_in(dk, 1), (BATCH * NEGS,),
                                   0, VOCAB).astype(jnp.int32)
    noise3 = noise_idx.reshape(NW, G, NEG_G)

    dots = _sc_dots(center.astype(jnp.int32), context.astype(jnp.int32),
                    noise3, mask_scale, W_center, W_context)

    loss = pl.pallas_call(
        _tc_loss_kernel,
        out_shape=jax.ShapeDtypeStruct((1, 1), jnp.float32),
        out_specs=pl.BlockSpec(memory_space=pltpu.SMEM),
    )(dots)
    return loss[0, 0]


# speedup vs baseline: 1.3357x; 1.3357x over previous
import functools

import jax
import jax.numpy as jnp
from jax import lax
from jax.experimental import pallas as pl
from jax.experimental.pallas import tpu as pltpu
from jax.experimental.pallas import tpu_sc as plsc

VOCAB = 1000000
EMB = 64
BATCH = 16384
NEGS = 15

NC = 2
NS = 16
L = 16
NW = NC * NS
BPW = BATCH // NW
GSZ = 32
G = BPW // GSZ
NEG_G = NEGS * GSZ
NCH = EMB // L


def _sc_dots_kernel(center_hbm, context_hbm, noise_hbm, mask_hbm,
                    wc_hbm, wx_hbm, out_hbm,
                    cidx_v, xidx_v, nidx_v,
                    cen0, ctx0, msk0, neg0, cen1, ctx1, msk1, neg1,
                    dots_v, sem0, sem1):
    wid = lax.axis_index("s") * NC + lax.axis_index("c")
    base = pl.multiple_of(wid * BPW, BPW)

    pltpu.sync_copy(center_hbm.at[pl.ds(base, BPW)], cidx_v)
    pltpu.sync_copy(context_hbm.at[pl.ds(base, BPW)], xidx_v)
    pltpu.sync_copy(noise_hbm.at[wid], nidx_v)

    bufs = ((cen0, ctx0, msk0, neg0, sem0), (cen1, ctx1, msk1, neg1, sem1))

    def issue(g, s):
        cen_v, ctx_v, msk_v, neg_v, sem = bufs[s]
        row0 = pl.multiple_of(g * GSZ, GSZ)
        pltpu.async_copy(wc_hbm.at[cidx_v.at[pl.ds(row0, GSZ)]], cen_v, sem)
        pltpu.async_copy(wx_hbm.at[xidx_v.at[pl.ds(row0, GSZ)]], ctx_v, sem)
        pltpu.async_copy(wx_hbm.at[nidx_v.at[g]], neg_v, sem)
        pltpu.async_copy(mask_hbm.at[pl.ds(base + row0, GSZ)], msk_v, sem)

    def drain(s):
        cen_v, ctx_v, msk_v, neg_v, sem = bufs[s]
        pltpu.make_async_copy(wc_hbm.at[pl.ds(0, GSZ)], cen_v, sem).wait()
        pltpu.make_async_copy(wx_hbm.at[pl.ds(0, GSZ)], ctx_v, sem).wait()
        pltpu.make_async_copy(wx_hbm.at[pl.ds(0, NEG_G)], neg_v, sem).wait()
        pltpu.make_async_copy(mask_hbm.at[pl.ds(0, GSZ)], msk_v, sem).wait()

    lane = lax.iota(jnp.int32, L)
    lane_mask = [lane == k for k in range(L)]

    def compute(g, s):
        cen_v, ctx_v, msk_v, neg_v, _ = bufs[s]
        row0 = pl.multiple_of(g * GSZ, GSZ)

        def body(b, carry):
            cm = [cen_v[b, pl.ds(c * L, L)] * msk_v[b, pl.ds(c * L, L)]
                  for c in range(NCH)]
            t = cm[0] * ctx_v[b, pl.ds(0, L)]
            for c in range(1, NCH):
                t = t + cm[c] * ctx_v[b, pl.ds(c * L, L)]
            acc = jnp.where(lane_mask[0], jnp.sum(t), 0.0)
            for j in range(NEGS):
                nrow = b * NEGS + j
                t = cm[0] * neg_v[nrow, pl.ds(0, L)]
                for c in range(1, NCH):
                    t = t + cm[c] * neg_v[nrow, pl.ds(c * L, L)]
                acc = jnp.where(lane_mask[1 + j], jnp.sum(t), acc)
            dots_v[row0 + b, :] = acc
            return carry

        lax.fori_loop(0, GSZ, body, 0)

    issue(0, 0)

    def pair(i, carry):
        g0 = pl.multiple_of(i * 2, 2)
        issue(g0 + 1, 1)
        drain(0)
        compute(g0, 0)

        @pl.when(g0 + 2 < G)
        def _():
            issue(g0 + 2, 0)

        drain(1)
        compute(g0 + 1, 1)
        return carry

    lax.fori_loop(0, G // 2, pair, 0)
    pltpu.sync_copy(dots_v, out_hbm.at[pl.ds(base, BPW)])


@functools.partial(
    pl.kernel,
    out_type=jax.ShapeDtypeStruct((BATCH, 1 + NEGS), jnp.float32),
    mesh=plsc.VectorSubcoreMesh(core_axis_name="c", subcore_axis_name="s"),
    compiler_params=pltpu.CompilerParams(needs_layout_passes=False,
                                         use_tc_tiling_on_sc=False),
    scratch_types=[
        pltpu.VMEM((BPW,), jnp.int32),
        pltpu.VMEM((BPW,), jnp.int32),
        pltpu.VMEM((G, NEG_G), jnp.int32),
        pltpu.VMEM((GSZ, EMB), jnp.float32),
        pltpu.VMEM((GSZ, EMB), jnp.float32),
        pltpu.VMEM((GSZ, EMB), jnp.float32),
        pltpu.VMEM((NEG_G, EMB), jnp.float32),
        pltpu.VMEM((GSZ, EMB), jnp.float32),
        pltpu.VMEM((GSZ, EMB), jnp.float32),
        pltpu.VMEM((GSZ, EMB), jnp.float32),
        pltpu.VMEM((NEG_G, EMB), jnp.float32),
        pltpu.VMEM((BPW, 1 + NEGS), jnp.float32),
        pltpu.SemaphoreType.DMA,
        pltpu.SemaphoreType.DMA,
    ],
)
def _sc_dots(center, context, noise, mask, wc, wx, out,
             cidx_v, xidx_v, nidx_v,
             cen0, ctx0, msk0, neg0, cen1, ctx1, msk1, neg1,
             dots_v, sem0, sem1):
    _sc_dots_kernel(center, context, noise, mask, wc, wx, out,
                    cidx_v, xidx_v, nidx_v,
                    cen0, ctx0, msk0, neg0, cen1, ctx1, msk1, neg1,
                    dots_v, sem0, sem1)


def _tc_loss_kernel(dots_ref, o_ref):
    x = dots_ref[...]
    col = lax.broadcasted_iota(jnp.int32, x.shape, 1)
    z = jnp.where(col == 0, x, -x)
    ls = jnp.minimum(z, 0.0) - jnp.log1p(jnp.exp(-jnp.abs(z)))
    o_ref[0, 0] = -jnp.sum(ls) / jnp.float32(BATCH)


def kernel(center, context, W_center, W_context):
    dk = jax.random.key(123)
    keep = jax.random.bernoulli(jax.random.fold_in(dk, 0), 0.9, (BATCH, EMB))
    mask_scale = jnp.where(keep, jnp.float32(1.0 / 0.9), jnp.float32(0.0))
    noise_idx = jax.random.randint(jax.random.fold_in(dk, 1), (BATCH * NEGS,),
                                   0, VOCAB).astype(jnp.int32)
    noise3 = noise_idx.reshape(NW, G, NEG_G)

    dots = _sc_dots(center.astype(jnp.int32), context.astype(jnp.int32),
                    noise3, mask_scale, W_center, W_context)

    loss = pl.pallas_call(
        _tc_loss_kernel,
        out_shape=jax.ShapeDtypeStruct((1, 1), jnp.float32),
        out_specs=pl.BlockSpec(memory_space=pltpu.SMEM),
    )(dots)
    return loss[0, 0]
